# baseline (speedup 1.0000x reference)
import functools

import jax
import jax.numpy as jnp
from jax import lax
from jax.experimental import pallas as pl
from jax.experimental.pallas import tpu as pltpu, tpu_sc as plsc

F32 = jnp.float32

_NC = 2
_NS = 16
_NW = _NC * _NS


def _node_body(x_ref, w_ref, as_ref, ad_ref, xl_ref, u_ref, v_ref, o_ref):
    i = pl.program_id(0)
    xl = jnp.dot(x_ref[...], w_ref[...], preferred_element_type=F32)
    xl_ref[...] = xl
    u = jnp.dot(xl, as_ref[...], preferred_element_type=F32)
    u_ref[...] = u
    v_ref[...] = jnp.dot(xl, ad_ref[...], preferred_element_type=F32)
    m = jnp.max(u, axis=0, keepdims=True)
    col = lax.broadcasted_iota(jnp.int32, (16, 16), 1)
    row = lax.broadcasted_iota(jnp.int32, (16, 16), 0)
    tmat = (col % 4 == row).astype(F32)
    mt = jnp.dot(m, tmat, preferred_element_type=F32)

    @pl.when(i == 0)
    def _():
        o_ref[...] = mt

    @pl.when(i > 0)
    def _():
        o_ref[...] = jnp.maximum(o_ref[...], mt)


def _edge_mlp_body(ea_ref, w1_ref, b1_ref, w2_ref, b2_ref,
                   a1_ref, ab1_ref, a2_ref, ab2_ref, pe_ref, lg_ref, mx_ref):
    i = pl.program_id(0)
    t = jnp.dot(ea_ref[...], w1_ref[...], preferred_element_type=F32)
    t = jnp.maximum(t + b1_ref[...], 0.0)
    pe = jnp.dot(t, w2_ref[...], preferred_element_type=F32) + b2_ref[...]
    pe_ref[...] = pe
    q = jnp.dot(pe, a1_ref[...], preferred_element_type=F32)
    q = jnp.maximum(q + ab1_ref[...], 0.0)
    lg = jnp.dot(q, a2_ref[...], preferred_element_type=F32) + ab2_ref[...]
    lg_ref[...] = lg.reshape(1, 1, lg.shape[0])
    m = jnp.max(lg).reshape(1, 1)

    @pl.when(i == 0)
    def _():
        mx_ref[...] = m

    @pl.when(i > 0)
    def _():
        mx_ref[...] = jnp.maximum(mx_ref[...], m)


def _expsum_body(lg_ref, m_ref, w_ref, s_ref):
    i = pl.program_id(0)
    w = jnp.exp(lg_ref[...] - m_ref[...])
    w_ref[...] = w
    s = jnp.sum(w).reshape(1, 1)

    @pl.when(i == 0)
    def _():
        s_ref[...] = s

    @pl.when(i > 0)
    def _():
        s_ref[...] = s_ref[...] + s


def _sc_conv_body(xl_hbm, src_hbm, dst_hbm, u_hbm, v_hbm, m16_hbm,
                  numer_hbm, denom_hbm,
                  sidx0, didx0, dscat0, rows0, ubuf0, vbuf0, aebuf0,
                  sidx1, didx1, dscat1, rows1, ubuf1, vbuf1, aebuf1,
                  m16buf,
                  semi0, semuv0, semr0, semsc0, semi1, semuv1, semr1, semsc1,
                  numer_sp, denom_sp,
                  *, n_nodes, e_per_w, k_chunk):
    c = lax.axis_index("c")
    s = lax.axis_index("s")
    wid = c * _NS + s
    n_stripe = n_nodes // _NS
    sbase = s * n_stripe
    kc = k_chunk

    slots = ((sidx0, didx0, rows0, ubuf0, vbuf0, aebuf0, semi0, semuv0, semr0,
              dscat0, semsc0),
             (sidx1, didx1, rows1, ubuf1, vbuf1, aebuf1, semi1, semuv1, semr1,
              dscat1, semsc1))

    def _zrow(k, _):
        for j in range(8):
            rows0[k, pl.ds(16 * j, 16)] = jnp.zeros((16,), F32)
        aebuf0[k, pl.ds(0, 16)] = jnp.zeros((16,), F32)
        aebuf1[k, pl.ds(0, 16)] = jnp.zeros((16,), F32)
        return 0

    lax.fori_loop(0, kc, _zrow, 0)

    nz_full = n_stripe // kc
    nz_rem = n_stripe - nz_full * kc

    def _zsp(j, _):
        pltpu.sync_copy(rows0, numer_sp.at[pl.ds(sbase + j * kc, kc)])
        pltpu.sync_copy(aebuf0, denom_sp.at[pl.ds(sbase + j * kc, kc)])
        return 0

    lax.fori_loop(0, nz_full, _zsp, 0)
    if nz_rem:
        pltpu.sync_copy(rows0.at[pl.ds(0, nz_rem)],
                        numer_sp.at[pl.ds(sbase + nz_full * kc, nz_rem)])
        pltpu.sync_copy(aebuf0.at[pl.ds(0, nz_rem)],
                        denom_sp.at[pl.ds(sbase + nz_full * kc, nz_rem)])

    pltpu.sync_copy(m16_hbm, m16buf)
    plsc.subcore_barrier()

    mvec = m16buf[0, :]
    lane = lax.iota(jnp.int32, 16)
    rseq = lax.shift_right_logical(lane, 2)
    cseq = lax.bitwise_and(lane, 3)

    ebase = wid * e_per_w
    n_chunks = e_per_w // kc

    def fire_idx(i, sl):
        base = ebase + i * kc
        pltpu.async_copy(src_hbm.at[pl.ds(base, kc)], sl[0], sl[6])
        pltpu.async_copy(dst_hbm.at[pl.ds(base, kc)], sl[1], sl[6])

    def wait_idx(sl):
        pltpu.make_async_copy(src_hbm.at[pl.ds(0, kc)], sl[0], sl[6]).wait()
        pltpu.make_async_copy(dst_hbm.at[pl.ds(0, kc)], sl[1], sl[6]).wait()

    def fire_gathers(sl):
        pltpu.async_copy(u_hbm.at[sl[0]], sl[3], sl[7])
        pltpu.async_copy(v_hbm.at[sl[1]], sl[4], sl[7])
        pltpu.async_copy(xl_hbm.at[sl[0]], sl[2], sl[8])

    def wait_uv(sl):
        pltpu.make_async_copy(u_hbm.at[pl.ds(0, kc)], sl[3], sl[7]).wait()
        pltpu.make_async_copy(v_hbm.at[pl.ds(0, kc)], sl[4], sl[7]).wait()

    def wait_rows(sl):
        pltpu.make_async_copy(xl_hbm.at[pl.ds(0, kc)], sl[2], sl[8]).wait()

    def fire_scatter(sl):
        def _cpi(j, _):
            sl[9][pl.ds(16 * j, 16)] = sl[1][pl.ds(16 * j, 16)]
            return 0

        lax.fori_loop(0, kc // 16, _cpi, 0)
        pltpu.async_copy(sl[2], numer_sp.at[sl[9]], sl[10], add=True)
        pltpu.async_copy(sl[5], denom_sp.at[sl[9]], sl[10], add=True)

    def wait_scatter(sl):
        pltpu.make_async_copy(sl[2], numer_sp.at[pl.ds(0, kc)], sl[10]).wait()
        pltpu.make_async_copy(sl[5], denom_sp.at[pl.ds(0, kc)], sl[10]).wait()

    def compute_ae(sl):
        ubuf, vbuf, aebuf = sl[3], sl[4], sl[5]

        def _ae(j, rj):
            ue = plsc.load_gather(ubuf, [rj, cseq])
            ve = plsc.load_gather(vbuf, [rj, cseq])
            su = ue + ve
            al = jnp.maximum(su, 0.2 * su)
            tb = mvec + ve
            bb = jnp.maximum(tb, 0.2 * tb)
            plsc.store_scatter(aebuf, [rj, cseq], jnp.exp(al - bb))
            return rj + 4

        lax.fori_loop(0, kc // 4, _ae, rseq)

    def scale_rows(sl):
        rows, aebuf = sl[2], sl[5]

        def _scale(k2, _):
            for b2 in range(2):
                k = 2 * k2 + b2
                for h in range(4):
                    ae_s = plsc.load_gather(
                        aebuf, [jnp.full((16,), k, jnp.int32),
                                jnp.full((16,), h, jnp.int32)])
                    for j2 in range(2):
                        off = 32 * h + 16 * j2
                        rows[k, pl.ds(off, 16)] = rows[k, pl.ds(off, 16)] * ae_s
            return 0

        lax.fori_loop(0, kc // 2, _scale, 0)

    pltpu.sync_copy(src_hbm.at[pl.ds(ebase, kc)], sidx0)
    pltpu.sync_copy(dst_hbm.at[pl.ds(ebase, kc)], didx0)
    fire_gathers(slots[0])
    fire_idx(1, slots[1])

    def _pair(m, _):
        for b in range(2):
            i2 = 2 * m + b
            sl = slots[b]
            nx = slots[1 - b]

            @pl.when(i2 < n_chunks)
            def _():
                wait_uv(sl)
                compute_ae(sl)
                wait_rows(sl)

                @pl.when(i2 > 0)
                def _():
                    wait_scatter(nx)

                @pl.when(i2 + 1 < n_chunks)
                def _():
                    wait_idx(nx)
                    fire_gathers(nx)

                scale_rows(sl)
                fire_scatter(sl)

                @pl.when(i2 + 2 < n_chunks)
                def _():
                    fire_idx(i2 + 2, sl)
        return 0

    lax.fori_loop(0, (n_chunks + 1) // 2, _pair, 0)
    wait_scatter(slots[(n_chunks - 1) % 2])
    plsc.subcore_barrier()

    pltpu.sync_copy(numer_sp.at[pl.ds(sbase, n_stripe)], numer_hbm.at[c, s])
    pltpu.sync_copy(denom_sp.at[pl.ds(sbase, n_stripe)], denom_hbm.at[c, s])


def _sc_agg_body(pe_hbm, src_hbm, w_hbm,
                 agg_hbm,
                 sidx0, rows0, wbuf0, sidx1, rows1, wbuf1,
                 sem0, sem1, semsc0, semsc1, agg_sp,
                 *, n_nodes, e_per_w, k_chunk):
    c = lax.axis_index("c")
    s = lax.axis_index("s")
    wid = c * _NS + s
    n_stripe = n_nodes // _NS
    sbase = s * n_stripe
    kc = k_chunk

    slots = ((sidx0, rows0, wbuf0, sem0, semsc0),
             (sidx1, rows1, wbuf1, sem1, semsc1))

    def _zrow(k, _):
        for j in range(8):
            rows0[k, pl.ds(16 * j, 16)] = jnp.zeros((16,), F32)
        return 0

    lax.fori_loop(0, kc, _zrow, 0)

    nz_full = n_stripe // kc
    nz_rem = n_stripe - nz_full * kc

    def _zagg(j, _):
        pltpu.sync_copy(rows0, agg_sp.at[pl.ds(sbase + j * kc, kc)])
        return 0

    lax.fori_loop(0, nz_full, _zagg, 0)
    if nz_rem:
        pltpu.sync_copy(rows0.at[pl.ds(0, nz_rem)],
                        agg_sp.at[pl.ds(sbase + nz_full * kc, nz_rem)])

    plsc.subcore_barrier()

    ebase = wid * e_per_w
    n_chunks = e_per_w // kc

    def fire_loads(i, sl):
        base = ebase + i * kc
        pltpu.async_copy(src_hbm.at[pl.ds(base, kc)], sl[0], sl[3])
        pltpu.async_copy(pe_hbm.at[pl.ds(base, kc)], sl[1], sl[3])
        pltpu.async_copy(w_hbm.at[pl.ds(base, kc)], sl[2], sl[3])

    def wait_loads(sl):
        pltpu.make_async_copy(src_hbm.at[pl.ds(0, kc)], sl[0], sl[3]).wait()
        pltpu.make_async_copy(pe_hbm.at[pl.ds(0, kc)], sl[1], sl[3]).wait()
        pltpu.make_async_copy(w_hbm.at[pl.ds(0, kc)], sl[2], sl[3]).wait()

    fire_loads(0, slots[0])

    def _pair(m, _):
        for b in range(2):
            i2 = 2 * m + b
            sl = slots[b]
            nx = slots[1 - b]

            @pl.when(i2 < n_chunks)
            def _():
                wait_loads(sl)

                @pl.when(i2 > 0)
                def _():
                    pltpu.make_async_copy(
                        nx[1], agg_sp.at[pl.ds(0, kc)], nx[4]).wait()

                @pl.when(i2 + 1 < n_chunks)
                def _():
                    fire_loads(i2 + 1, nx)

                rows, wbuf = sl[1], sl[2]

                def _scale(k2, _):
                    for b2 in range(2):
                        k = 2 * k2 + b2
                        w_s = plsc.load_gather(
                            wbuf, [jnp.full((16,), k, jnp.int32)])
                        for j in range(8):
                            rows[k, pl.ds(16 * j, 16)] = (
                                rows[k, pl.ds(16 * j, 16)] * w_s)
                    return 0

                lax.fori_loop(0, kc // 2, _scale, 0)
                pltpu.async_copy(rows, agg_sp.at[sl[0]], sl[4], add=True)
        return 0

    lax.fori_loop(0, (n_chunks + 1) // 2, _pair, 0)
    pltpu.make_async_copy(slots[(n_chunks - 1) % 2][1],
                          agg_sp.at[pl.ds(0, kc)],
                          slots[(n_chunks - 1) % 2][4]).wait()
    plsc.subcore_barrier()

    pltpu.sync_copy(agg_sp.at[pl.ds(sbase, n_stripe)], agg_hbm.at[c, s])


def _sc_fused_body(xl_hbm, src_hbm, dst_hbm, u_hbm, v_hbm, m16_hbm,
                   pe_hbm, w_hbm,
                   numer_hbm, denom_hbm, agg_hbm,
                   sidx0, didx0, dscat0, rows0, ubuf0, vbuf0, aebuf0,
                   sidx1, didx1, dscat1, rows1, ubuf1, vbuf1, aebuf1,
                   m16buf, wbuf0, wbuf1,
                   semi0, semuv0, semr0, semsc0, semi1, semuv1, semr1, semsc1,
                   numer_sp, denom_sp,
                   *, n_nodes, e_per_w, k_chunk):
    _sc_conv_body(xl_hbm, src_hbm, dst_hbm, u_hbm, v_hbm, m16_hbm,
                  numer_hbm, denom_hbm,
                  sidx0, didx0, dscat0, rows0, ubuf0, vbuf0, aebuf0,
                  sidx1, didx1, dscat1, rows1, ubuf1, vbuf1, aebuf1,
                  m16buf,
                  semi0, semuv0, semr0, semsc0, semi1, semuv1, semr1, semsc1,
                  numer_sp, denom_sp,
                  n_nodes=n_nodes, e_per_w=e_per_w, k_chunk=k_chunk)
    _sc_agg_body(pe_hbm, src_hbm, w_hbm, agg_hbm,
                 sidx0, rows0, wbuf0, sidx1, rows1, wbuf1,
                 semi0, semi1, semsc0, semsc1, numer_sp,
                 n_nodes=n_nodes, e_per_w=e_per_w, k_chunk=k_chunk)


def _combine_body(numer_ref, denom_ref, agg_ref, xl_ref, u_ref, v_ref,
                  m_ref, bg_ref, s_ref, r_ref, b_ref, fcw_ref, fcb_ref,
                  out_ref, acc_ref, cnt_ref, *, n_steps):
    i = pl.program_id(0)
    u = u_ref[:, :4]
    v = v_ref[:, :4]
    mx = m_ref[:, :4]
    tb = mx + v
    bb = jnp.maximum(tb, 0.2 * tb)
    su = u + v
    al = jnp.maximum(su, 0.2 * su)
    aes = jnp.exp(al - bb)
    dn = denom_ref[0, :, :4] + denom_ref[1, :, :4] + aes
    r = r_ref[...]
    xl = xl_ref[...]
    nm = (numer_ref[0] + numer_ref[1]
          + jnp.dot(aes, r, preferred_element_type=F32) * xl)
    dnr = jnp.dot(dn, r, preferred_element_type=F32)
    conv = nm / dnr + bg_ref[...]
    h = jnp.where(conv > 0, conv, jnp.exp(jnp.minimum(conv, 0.0)) - 1.0)
    h = h + (agg_ref[0] + agg_ref[1]) / s_ref[...]

    g8 = lax.iota(jnp.int32, 8)[None, :]
    m8 = (b_ref[...] == g8).astype(F32)
    upd = lax.dot_general(m8, h, (((0,), (0,)), ((), ())),
                          preferred_element_type=F32)
    updc = jnp.sum(m8, axis=0, keepdims=True)

    @pl.when(i == 0)
    def _():
        acc_ref[...] = upd
        cnt_ref[...] = updc

    @pl.when(i > 0)
    def _():
        acc_ref[...] = acc_ref[...] + upd
        cnt_ref[...] = cnt_ref[...] + updc

    @pl.when(i == n_steps - 1)
    def _():
        cnt = jnp.maximum(cnt_ref[...], 1.0)
        ii = lax.broadcasted_iota(jnp.int32, (8, 8), 0)
        jj = lax.broadcasted_iota(jnp.int32, (8, 8), 1)
        dmat = jnp.where(ii == jj, 1.0 / cnt, 0.0)
        pooled = jnp.dot(dmat, acc_ref[...], preferred_element_type=F32)
        out_ref[...] = (jnp.dot(pooled, fcw_ref[...],
                                preferred_element_type=F32) + fcb_ref[...])


def _run_sc_fused(xl, src, dst, u, v, m16, pe, w_flat, n, e, hc, heads):
    e_per_w = e // _NW
    k_chunk = 80
    mesh = plsc.VectorSubcoreMesh(core_axis_name="c", subcore_axis_name="s",
                                  num_cores=_NC, num_subcores=_NS)
    numer, denom, agg = pl.kernel(
        functools.partial(_sc_fused_body, n_nodes=n, e_per_w=e_per_w,
                          k_chunk=k_chunk),
        out_type=[
            jax.ShapeDtypeStruct((_NC, _NS, n // _NS, hc), F32),
            jax.ShapeDtypeStruct((_NC, _NS, n // _NS, 16), F32),
            jax.ShapeDtypeStruct((_NC, _NS, n // _NS, hc), F32),
        ],
        mesh=mesh,
        compiler_params=pltpu.CompilerParams(needs_layout_passes=False,
                                             use_tc_tiling_on_sc=False),
        scratch_types=(
            [pltpu.VMEM((k_chunk,), jnp.int32),
             pltpu.VMEM((k_chunk,), jnp.int32),
             pltpu.VMEM((k_chunk,), jnp.int32),
             pltpu.VMEM((k_chunk, hc), F32),
             pltpu.VMEM((k_chunk, 16), F32),
             pltpu.VMEM((k_chunk, 16), F32),
             pltpu.VMEM((k_chunk, 16), F32)] * 2
            + [pltpu.VMEM((1, 16), F32),
               pltpu.VMEM((k_chunk,), F32),
               pltpu.VMEM((k_chunk,), F32)]
            + [pltpu.SemaphoreType.DMA] * 8
            + [pltpu.VMEM_SHARED((n, hc), F32),
               pltpu.VMEM_SHARED((n, 16), F32)]),
    )(xl, src, dst, u, v, m16, pe, w_flat)
    return (numer.reshape(_NC, n, hc), denom.reshape(_NC, n, 16),
            agg.reshape(_NC, n, hc))


def kernel(x, edge_index, edge_attr, batch, W_gat, att_src, att_dst, b_gat,
           em_w1, em_b1, em_w2, em_b2, ea_w1, ea_b1, ea_w2, ea_b2, fc_w, fc_b):
    n, df = x.shape
    e = edge_attr.shape[0]
    de = edge_attr.shape[1]
    hc = W_gat.shape[1]
    heads = att_src.shape[1]
    ch = att_src.shape[2]
    hid = em_w1.shape[1]
    hid2 = ea_w1.shape[1]
    out_d = fc_w.shape[1]
    g = 8

    eyeh = jnp.eye(heads, dtype=F32)
    a_s = (eyeh[:, None, :] * att_src[0][:, :, None]).reshape(hc, heads)
    a_d = (eyeh[:, None, :] * att_dst[0][:, :, None]).reshape(hc, heads)
    a_s = jnp.pad(a_s, ((0, 0), (0, 16 - heads)))
    a_d = jnp.pad(a_d, ((0, 0), (0, 16 - heads)))
    rmat = jnp.repeat(eyeh, ch, axis=1)

    nt = 400
    n_steps = n // nt

    xl = pl.pallas_call(
        _node_body,
        grid=(n_steps,),
        in_specs=[
            pl.BlockSpec((nt, df), lambda i: (i, 0)),
            pl.BlockSpec((df, hc), lambda i: (0, 0)),
            pl.BlockSpec((hc, 16), lambda i: (0, 0)),
            pl.BlockSpec((hc, 16), lambda i: (0, 0)),
        ],
        out_specs=[
            pl.BlockSpec((nt, hc), lambda i: (i, 0)),
            pl.BlockSpec((nt, 16), lambda i: (i, 0)),
            pl.BlockSpec((nt, 16), lambda i: (i, 0)),
            pl.BlockSpec((1, 16), lambda i: (0, 0)),
        ],
        out_shape=[
            jax.ShapeDtypeStruct((n, hc), F32),
            jax.ShapeDtypeStruct((n, 16), F32),
            jax.ShapeDtypeStruct((n, 16), F32),
            jax.ShapeDtypeStruct((1, 16), F32),
        ],
    )(x, W_gat, a_s, a_d)
    xl, u, v, m16 = xl

    et = 1280
    e_steps = e // et
    pe, logits, lmax = pl.pallas_call(
        _edge_mlp_body,
        grid=(e_steps,),
        in_specs=[
            pl.BlockSpec((et, de), lambda i: (i, 0)),
            pl.BlockSpec((de, hid), lambda i: (0, 0)),
            pl.BlockSpec((1, hid), lambda i: (0, 0)),
            pl.BlockSpec((hid, hid), lambda i: (0, 0)),
            pl.BlockSpec((1, hid), lambda i: (0, 0)),
            pl.BlockSpec((hid, hid2), lambda i: (0, 0)),
            pl.BlockSpec((1, hid2), lambda i: (0, 0)),
            pl.BlockSpec((hid2, 1), lambda i: (0, 0)),
            pl.BlockSpec((1, 1), lambda i: (0, 0)),
        ],
        out_specs=[
            pl.BlockSpec((et, hid), lambda i: (i, 0)),
            pl.BlockSpec((1, 1, et), lambda i: (i, 0, 0)),
            pl.BlockSpec((1, 1), lambda i: (0, 0)),
        ],
        out_shape=[
            jax.ShapeDtypeStruct((e, hid), F32),
            jax.ShapeDtypeStruct((e_steps, 1, et), F32),
            jax.ShapeDtypeStruct((1, 1), F32),
        ],
    )(edge_attr, em_w1, em_b1.reshape(1, hid), em_w2, em_b2.reshape(1, hid),
      ea_w1, ea_b1.reshape(1, hid2), ea_w2, ea_b2.reshape(1, 1))

    w_e, s_sum = pl.pallas_call(
        _expsum_body,
        grid=(1,),
        in_specs=[
            pl.BlockSpec((e_steps, et), lambda i: (0, 0)),
            pl.BlockSpec((1, 1), lambda i: (0, 0)),
        ],
        out_specs=[
            pl.BlockSpec((e_steps, et), lambda i: (0, 0)),
            pl.BlockSpec((1, 1), lambda i: (0, 0)),
        ],
        out_shape=[
            jax.ShapeDtypeStruct((e_steps, et), F32),
            jax.ShapeDtypeStruct((1, 1), F32),
        ],
    )(logits.reshape(e_steps, et), lmax)

    src = edge_index[0]
    dst = edge_index[1]
    numer, denom, agg = _run_sc_fused(xl, src, dst, u, v, m16, pe,
                                      w_e.reshape(e), n, e, hc, heads)

    out = pl.pallas_call(
        functools.partial(_combine_body, n_steps=n_steps),
        grid=(n_steps,),
        in_specs=[
            pl.BlockSpec((_NC, nt, hc), lambda i: (0, i, 0)),
            pl.BlockSpec((_NC, nt, 16), lambda i: (0, i, 0)),
            pl.BlockSpec((_NC, nt, hc), lambda i: (0, i, 0)),
            pl.BlockSpec((nt, hc), lambda i: (i, 0)),
            pl.BlockSpec((nt, 16), lambda i: (i, 0)),
            pl.BlockSpec((nt, 16), lambda i: (i, 0)),
            pl.BlockSpec((1, 4 * heads), lambda i: (0, 0)),
            pl.BlockSpec((1, hc), lambda i: (0, 0)),
            pl.BlockSpec((1, 1), lambda i: (0, 0)),
            pl.BlockSpec((heads, hc), lambda i: (0, 0)),
            pl.BlockSpec((nt, 1), lambda i: (i, 0)),
            pl.BlockSpec((hc, out_d), lambda i: (0, 0)),
            pl.BlockSpec((1, out_d), lambda i: (0, 0)),
        ],
        out_specs=pl.BlockSpec((g, out_d), lambda i: (0, 0)),
        out_shape=jax.ShapeDtypeStruct((g, out_d), F32),
        scratch_shapes=[
            pltpu.VMEM((g, hc), F32),
            pltpu.VMEM((1, g), F32),
        ],
    )(numer, denom, agg, xl, u, v, m16, b_gat.reshape(1, hc), s_sum, rmat,
      batch.reshape(n, 1), fc_w, fc_b.reshape(1, out_d))

    return out.reshape(-1)


# speedup vs baseline: 1.4343x; 1.4343x over previous
import functools

import jax
import jax.numpy as jnp
from jax import lax
from jax.experimental import pallas as pl
from jax.experimental.pallas import tpu as pltpu, tpu_sc as plsc

F32 = jnp.float32

_NC = 2
_NS = 16
_NW = _NC * _NS


def _node_body(x_ref, w_ref, as_ref, ad_ref, xl_ref, u_ref, v_ref, o_ref):
    i = pl.program_id(0)
    xl = jnp.dot(x_ref[...], w_ref[...], preferred_element_type=F32)
    xl_ref[...] = xl
    u = jnp.dot(xl, as_ref[...], preferred_element_type=F32)
    u_ref[...] = u
    v_ref[...] = jnp.dot(xl, ad_ref[...], preferred_element_type=F32)
    m = jnp.max(u, axis=0, keepdims=True)
    col = lax.broadcasted_iota(jnp.int32, (16, 16), 1)
    row = lax.broadcasted_iota(jnp.int32, (16, 16), 0)
    tmat = (col % 4 == row).astype(F32)
    mt = jnp.dot(m, tmat, preferred_element_type=F32)

    @pl.when(i == 0)
    def _():
        o_ref[...] = mt

    @pl.when(i > 0)
    def _():
        o_ref[...] = jnp.maximum(o_ref[...], mt)


def _edge_mlp_body(ea_ref, w1_ref, b1_ref, w2_ref, b2_ref,
                   a1_ref, ab1_ref, a2_ref, ab2_ref, pe_ref, lg_ref, mx_ref):
    i = pl.program_id(0)
    t = jnp.dot(ea_ref[...], w1_ref[...], preferred_element_type=F32)
    t = jnp.maximum(t + b1_ref[...], 0.0)
    pe = jnp.dot(t, w2_ref[...], preferred_element_type=F32) + b2_ref[...]
    pe_ref[...] = pe
    q = jnp.dot(pe, a1_ref[...], preferred_element_type=F32)
    q = jnp.maximum(q + ab1_ref[...], 0.0)
    lg = jnp.dot(q, a2_ref[...], preferred_element_type=F32) + ab2_ref[...]
    lg_ref[...] = lg.reshape(1, 1, lg.shape[0])
    m = jnp.max(lg).reshape(1, 1)

    @pl.when(i == 0)
    def _():
        mx_ref[...] = m

    @pl.when(i > 0)
    def _():
        mx_ref[...] = jnp.maximum(mx_ref[...], m)


def _expsum_body(lg_ref, m_ref, w_ref, s_ref):
    i = pl.program_id(0)
    w = jnp.exp(lg_ref[...] - m_ref[...])
    w_ref[...] = w
    s = jnp.sum(w).reshape(1, 1)

    @pl.when(i == 0)
    def _():
        s_ref[...] = s

    @pl.when(i > 0)
    def _():
        s_ref[...] = s_ref[...] + s


def _sc_conv_body(xl_hbm, src_hbm, dst_hbm, u_hbm, v_hbm, m16_hbm,
                  numer_hbm, denom_hbm,
                  sidx0, didx0, dscat0, rows0, ubuf0, vbuf0, aebuf0,
                  sidx1, didx1, dscat1, rows1, ubuf1, vbuf1, aebuf1,
                  m16buf,
                  semi0, semuv0, semr0, semsc0, semi1, semuv1, semr1, semsc1,
                  numer_sp, denom_sp,
                  *, n_nodes, e_per_w, k_chunk):
    c = lax.axis_index("c")
    s = lax.axis_index("s")
    wid = c * _NS + s
    n_stripe = n_nodes // _NS
    sbase = s * n_stripe
    kc = k_chunk

    slots = ((sidx0, didx0, rows0, ubuf0, vbuf0, aebuf0, semi0, semuv0, semr0,
              dscat0, semsc0),
             (sidx1, didx1, rows1, ubuf1, vbuf1, aebuf1, semi1, semuv1, semr1,
              dscat1, semsc1))

    def _zrow(k, _):
        for j in range(8):
            rows0[k, pl.ds(16 * j, 16)] = jnp.zeros((16,), F32)
        aebuf0[k, pl.ds(0, 16)] = jnp.zeros((16,), F32)
        aebuf1[k, pl.ds(0, 16)] = jnp.zeros((16,), F32)
        return 0

    lax.fori_loop(0, kc, _zrow, 0)

    nz_full = n_stripe // kc
    nz_rem = n_stripe - nz_full * kc

    def _zsp(j, _):
        pltpu.sync_copy(rows0, numer_sp.at[pl.ds(sbase + j * kc, kc)])
        pltpu.sync_copy(aebuf0, denom_sp.at[pl.ds(sbase + j * kc, kc)])
        return 0

    lax.fori_loop(0, nz_full, _zsp, 0)
    if nz_rem:
        pltpu.sync_copy(rows0.at[pl.ds(0, nz_rem)],
                        numer_sp.at[pl.ds(sbase + nz_full * kc, nz_rem)])
        pltpu.sync_copy(aebuf0.at[pl.ds(0, nz_rem)],
                        denom_sp.at[pl.ds(sbase + nz_full * kc, nz_rem)])

    pltpu.sync_copy(m16_hbm, m16buf)
    plsc.subcore_barrier()

    mvec = m16buf[0, :]
    lane = lax.iota(jnp.int32, 16)
    rseq = lax.shift_right_logical(lane, 2)
    cseq = lax.bitwise_and(lane, 3)

    ebase = wid * e_per_w
    n_chunks = e_per_w // kc

    def fire_idx(i, sl):
        base = ebase + i * kc
        pltpu.async_copy(src_hbm.at[pl.ds(base, kc)], sl[0], sl[6])
        pltpu.async_copy(dst_hbm.at[pl.ds(base, kc)], sl[1], sl[6])

    def wait_idx(sl):
        pltpu.make_async_copy(src_hbm.at[pl.ds(0, kc)], sl[0], sl[6]).wait()
        pltpu.make_async_copy(dst_hbm.at[pl.ds(0, kc)], sl[1], sl[6]).wait()

    def fire_gathers(sl):
        pltpu.async_copy(u_hbm.at[sl[0]], sl[3], sl[7])
        pltpu.async_copy(v_hbm.at[sl[1]], sl[4], sl[7])
        pltpu.async_copy(xl_hbm.at[sl[0]], sl[2], sl[8])

    def wait_uv(sl):
        pltpu.make_async_copy(u_hbm.at[pl.ds(0, kc)], sl[3], sl[7]).wait()
        pltpu.make_async_copy(v_hbm.at[pl.ds(0, kc)], sl[4], sl[7]).wait()

    def wait_rows(sl):
        pltpu.make_async_copy(xl_hbm.at[pl.ds(0, kc)], sl[2], sl[8]).wait()

    def fire_scatter(sl):
        def _cpi(j, _):
            sl[9][pl.ds(16 * j, 16)] = sl[1][pl.ds(16 * j, 16)]
            return 0

        lax.fori_loop(0, kc // 16, _cpi, 0)
        pltpu.async_copy(sl[2], numer_sp.at[sl[9]], sl[10], add=True)
        pltpu.async_copy(sl[5], denom_sp.at[sl[9]], sl[10], add=True)

    def wait_scatter(sl):
        pltpu.make_async_copy(sl[2], numer_sp.at[pl.ds(0, kc)], sl[10]).wait()
        pltpu.make_async_copy(sl[5], denom_sp.at[pl.ds(0, kc)], sl[10]).wait()

    def compute_ae(sl):
        ubuf, vbuf, aebuf = sl[3], sl[4], sl[5]

        def _ae(j, rj):
            ue = plsc.load_gather(ubuf, [rj, cseq])
            ve = plsc.load_gather(vbuf, [rj, cseq])
            su = ue + ve
            al = jnp.maximum(su, 0.2 * su)
            tb = mvec + ve
            bb = jnp.maximum(tb, 0.2 * tb)
            plsc.store_scatter(aebuf, [rj, cseq], jnp.exp(al - bb))
            return rj + 4

        lax.fori_loop(0, kc // 4, _ae, rseq)

    def scale_rows(sl):
        rows, aebuf = sl[2], sl[5]

        def _scale(k2, _):
            for b2 in range(2):
                k = 2 * k2 + b2
                for h in range(4):
                    ae_s = plsc.load_gather(
                        aebuf, [jnp.full((16,), k, jnp.int32),
                                jnp.full((16,), h, jnp.int32)])
                    for j2 in range(2):
                        off = 32 * h + 16 * j2
                        rows[k, pl.ds(off, 16)] = rows[k, pl.ds(off, 16)] * ae_s
            return 0

        lax.fori_loop(0, kc // 2, _scale, 0)

    pltpu.sync_copy(src_hbm.at[pl.ds(ebase, kc)], sidx0)
    pltpu.sync_copy(dst_hbm.at[pl.ds(ebase, kc)], didx0)
    fire_gathers(slots[0])
    fire_idx(1, slots[1])

    def _pair(m, _):
        for b in range(2):
            i2 = 2 * m + b
            sl = slots[b]
            nx = slots[1 - b]

            @pl.when(i2 < n_chunks)
            def _():
                wait_uv(sl)
                compute_ae(sl)
                wait_rows(sl)

                @pl.when(i2 > 0)
                def _():
                    wait_scatter(nx)

                @pl.when(i2 + 1 < n_chunks)
                def _():
                    wait_idx(nx)
                    fire_gathers(nx)

                scale_rows(sl)
                fire_scatter(sl)

                @pl.when(i2 + 2 < n_chunks)
                def _():
                    fire_idx(i2 + 2, sl)
        return 0

    lax.fori_loop(0, (n_chunks + 1) // 2, _pair, 0)
    wait_scatter(slots[(n_chunks - 1) % 2])
    plsc.subcore_barrier()

    pltpu.sync_copy(numer_sp.at[pl.ds(sbase, n_stripe)], numer_hbm.at[c, s])
    pltpu.sync_copy(denom_sp.at[pl.ds(sbase, n_stripe)], denom_hbm.at[c, s])


def _sc_agg_body(pe_hbm, src_hbm, w_hbm,
                 agg_hbm,
                 sidx0, rows0, wbuf0, sidx1, rows1, wbuf1,
                 sem0, sem1, semsc0, semsc1, agg_sp,
                 *, n_nodes, e_per_w, k_chunk):
    c = lax.axis_index("c")
    s = lax.axis_index("s")
    wid = c * _NS + s
    n_stripe = n_nodes // _NS
    sbase = s * n_stripe
    kc = k_chunk

    slots = ((sidx0, rows0, wbuf0, sem0, semsc0),
             (sidx1, rows1, wbuf1, sem1, semsc1))

    def _zrow(k, _):
        for j in range(8):
            rows0[k, pl.ds(16 * j, 16)] = jnp.zeros((16,), F32)
        return 0

    lax.fori_loop(0, kc, _zrow, 0)

    nz_full = n_stripe // kc
    nz_rem = n_stripe - nz_full * kc

    def _zagg(j, _):
        pltpu.sync_copy(rows0, agg_sp.at[pl.ds(sbase + j * kc, kc)])
        return 0

    lax.fori_loop(0, nz_full, _zagg, 0)
    if nz_rem:
        pltpu.sync_copy(rows0.at[pl.ds(0, nz_rem)],
                        agg_sp.at[pl.ds(sbase + nz_full * kc, nz_rem)])

    plsc.subcore_barrier()

    ebase = wid * e_per_w
    n_chunks = e_per_w // kc

    def fire_loads(i, sl):
        base = ebase + i * kc
        pltpu.async_copy(src_hbm.at[pl.ds(base, kc)], sl[0], sl[3])
        pltpu.async_copy(pe_hbm.at[pl.ds(base, kc)], sl[1], sl[3])
        pltpu.async_copy(w_hbm.at[pl.ds(base, kc)], sl[2], sl[3])

    def wait_loads(sl):
        pltpu.make_async_copy(src_hbm.at[pl.ds(0, kc)], sl[0], sl[3]).wait()
        pltpu.make_async_copy(pe_hbm.at[pl.ds(0, kc)], sl[1], sl[3]).wait()
        pltpu.make_async_copy(w_hbm.at[pl.ds(0, kc)], sl[2], sl[3]).wait()

    fire_loads(0, slots[0])

    def _pair(m, _):
        for b in range(2):
            i2 = 2 * m + b
            sl = slots[b]
            nx = slots[1 - b]

            @pl.when(i2 < n_chunks)
            def _():
                wait_loads(sl)

                @pl.when(i2 > 0)
                def _():
                    pltpu.make_async_copy(
                        nx[1], agg_sp.at[pl.ds(0, kc)], nx[4]).wait()

                @pl.when(i2 + 1 < n_chunks)
                def _():
                    fire_loads(i2 + 1, nx)

                rows, wbuf = sl[1], sl[2]

                def _scale(k2, _):
                    for b2 in range(2):
                        k = 2 * k2 + b2
                        w_s = plsc.load_gather(
                            wbuf, [jnp.full((16,), k, jnp.int32)])
                        for j in range(8):
                            rows[k, pl.ds(16 * j, 16)] = (
                                rows[k, pl.ds(16 * j, 16)] * w_s)
                    return 0

                lax.fori_loop(0, kc // 2, _scale, 0)
                pltpu.async_copy(rows, agg_sp.at[sl[0]], sl[4], add=True)
        return 0

    lax.fori_loop(0, (n_chunks + 1) // 2, _pair, 0)
    pltpu.make_async_copy(slots[(n_chunks - 1) % 2][1],
                          agg_sp.at[pl.ds(0, kc)],
                          slots[(n_chunks - 1) % 2][4]).wait()
    plsc.subcore_barrier()

    pltpu.sync_copy(agg_sp.at[pl.ds(sbase, n_stripe)], agg_hbm.at[c, s])


def _combine_body(numer_ref, denom_ref, agg_ref, xl_ref, u_ref, v_ref,
                  m_ref, bg_ref, s_ref, r_ref, b_ref, fcw_ref, fcb_ref,
                  out_ref, acc_ref, cnt_ref, *, n_steps):
    i = pl.program_id(0)
    u = u_ref[:, :4]
    v = v_ref[:, :4]
    mx = m_ref[:, :4]
    tb = mx + v
    bb = jnp.maximum(tb, 0.2 * tb)
    su = u + v
    al = jnp.maximum(su, 0.2 * su)
    aes = jnp.exp(al - bb)
    dn = denom_ref[0, :, :4] + denom_ref[1, :, :4] + aes
    r = r_ref[...]
    xl = xl_ref[...]
    nm = (numer_ref[0] + numer_ref[1]
          + jnp.dot(aes, r, preferred_element_type=F32) * xl)
    dnr = jnp.dot(dn, r, preferred_element_type=F32)
    conv = nm / dnr + bg_ref[...]
    h = jnp.where(conv > 0, conv, jnp.exp(jnp.minimum(conv, 0.0)) - 1.0)
    h = h + (agg_ref[0] + agg_ref[1]) / s_ref[...]

    g8 = lax.iota(jnp.int32, 8)[None, :]
    m8 = (b_ref[...] == g8).astype(F32)
    upd = lax.dot_general(m8, h, (((0,), (0,)), ((), ())),
                          preferred_element_type=F32)
    updc = jnp.sum(m8, axis=0, keepdims=True)

    @pl.when(i == 0)
    def _():
        acc_ref[...] = upd
        cnt_ref[...] = updc

    @pl.when(i > 0)
    def _():
        acc_ref[...] = acc_ref[...] + upd
        cnt_ref[...] = cnt_ref[...] + updc

    @pl.when(i == n_steps - 1)
    def _():
        cnt = jnp.maximum(cnt_ref[...], 1.0)
        ii = lax.broadcasted_iota(jnp.int32, (8, 8), 0)
        jj = lax.broadcasted_iota(jnp.int32, (8, 8), 1)
        dmat = jnp.where(ii == jj, 1.0 / cnt, 0.0)
        pooled = jnp.dot(dmat, acc_ref[...], preferred_element_type=F32)
        out_ref[...] = (jnp.dot(pooled, fcw_ref[...],
                                preferred_element_type=F32) + fcb_ref[...])


def _run_sc_conv(xl, src, dst, u, v, m16, n, e, hc, heads):
    e_per_w = e // _NW
    k_chunk = 80
    mesh = plsc.VectorSubcoreMesh(core_axis_name="c", subcore_axis_name="s",
                                  num_cores=_NC, num_subcores=_NS)
    numer, denom = pl.kernel(
        functools.partial(_sc_conv_body, n_nodes=n, e_per_w=e_per_w,
                          k_chunk=k_chunk),
        out_type=[
            jax.ShapeDtypeStruct((_NC, _NS, n // _NS, hc), F32),
            jax.ShapeDtypeStruct((_NC, _NS, n // _NS, 16), F32),
        ],
        mesh=mesh,
        compiler_params=pltpu.CompilerParams(needs_layout_passes=False,
                                             use_tc_tiling_on_sc=False),
        scratch_types=(
            [pltpu.VMEM((k_chunk,), jnp.int32),
             pltpu.VMEM((k_chunk,), jnp.int32),
             pltpu.VMEM((k_chunk,), jnp.int32),
             pltpu.VMEM((k_chunk, hc), F32),
             pltpu.VMEM((k_chunk, 16), F32),
             pltpu.VMEM((k_chunk, 16), F32),
             pltpu.VMEM((k_chunk, 16), F32)] * 2
            + [pltpu.VMEM((1, 16), F32)]
            + [pltpu.SemaphoreType.DMA] * 8
            + [pltpu.VMEM_SHARED((n, hc), F32),
               pltpu.VMEM_SHARED((n, 16), F32)]),
    )(xl, src, dst, u, v, m16)
    return numer.reshape(_NC, n, hc), denom.reshape(_NC, n, 16)


def _run_sc_agg(pe, src, w_flat, n, e, hc):
    e_per_w = e // _NW
    k_chunk = 80
    mesh = plsc.VectorSubcoreMesh(core_axis_name="c", subcore_axis_name="s",
                                  num_cores=_NC, num_subcores=_NS)
    agg = pl.kernel(
        functools.partial(_sc_agg_body, n_nodes=n, e_per_w=e_per_w,
                          k_chunk=k_chunk),
        out_type=jax.ShapeDtypeStruct((_NC, _NS, n // _NS, hc), F32),
        mesh=mesh,
        compiler_params=pltpu.CompilerParams(needs_layout_passes=False,
                                             use_tc_tiling_on_sc=False),
        scratch_types=(
            [pltpu.VMEM((k_chunk,), jnp.int32),
             pltpu.VMEM((k_chunk, hc), F32),
             pltpu.VMEM((k_chunk,), F32)] * 2
            + [pltpu.SemaphoreType.DMA] * 4
            + [pltpu.VMEM_SHARED((n, hc), F32)]),
    )(pe, src, w_flat)
    return agg.reshape(_NC, n, hc)


def kernel(x, edge_index, edge_attr, batch, W_gat, att_src, att_dst, b_gat,
           em_w1, em_b1, em_w2, em_b2, ea_w1, ea_b1, ea_w2, ea_b2, fc_w, fc_b):
    n, df = x.shape
    e = edge_attr.shape[0]
    de = edge_attr.shape[1]
    hc = W_gat.shape[1]
    heads = att_src.shape[1]
    ch = att_src.shape[2]
    hid = em_w1.shape[1]
    hid2 = ea_w1.shape[1]
    out_d = fc_w.shape[1]
    g = 8

    eyeh = jnp.eye(heads, dtype=F32)
    a_s = (eyeh[:, None, :] * att_src[0][:, :, None]).reshape(hc, heads)
    a_d = (eyeh[:, None, :] * att_dst[0][:, :, None]).reshape(hc, heads)
    a_s = jnp.pad(a_s, ((0, 0), (0, 16 - heads)))
    a_d = jnp.pad(a_d, ((0, 0), (0, 16 - heads)))
    rmat = jnp.repeat(eyeh, ch, axis=1)

    nt = 400
    n_steps = n // nt

    xl = pl.pallas_call(
        _node_body,
        grid=(n_steps,),
        in_specs=[
            pl.BlockSpec((nt, df), lambda i: (i, 0)),
            pl.BlockSpec((df, hc), lambda i: (0, 0)),
            pl.BlockSpec((hc, 16), lambda i: (0, 0)),
            pl.BlockSpec((hc, 16), lambda i: (0, 0)),
        ],
        out_specs=[
            pl.BlockSpec((nt, hc), lambda i: (i, 0)),
            pl.BlockSpec((nt, 16), lambda i: (i, 0)),
            pl.BlockSpec((nt, 16), lambda i: (i, 0)),
            pl.BlockSpec((1, 16), lambda i: (0, 0)),
        ],
        out_shape=[
            jax.ShapeDtypeStruct((n, hc), F32),
            jax.ShapeDtypeStruct((n, 16), F32),
            jax.ShapeDtypeStruct((n, 16), F32),
            jax.ShapeDtypeStruct((1, 16), F32),
        ],
    )(x, W_gat, a_s, a_d)
    xl, u, v, m16 = xl

    et = 1280
    e_steps = e // et
    pe, logits, lmax = pl.pallas_call(
        _edge_mlp_body,
        grid=(e_steps,),
        in_specs=[
            pl.BlockSpec((et, de), lambda i: (i, 0)),
            pl.BlockSpec((de, hid), lambda i: (0, 0)),
            pl.BlockSpec((1, hid), lambda i: (0, 0)),
            pl.BlockSpec((hid, hid), lambda i: (0, 0)),
            pl.BlockSpec((1, hid), lambda i: (0, 0)),
            pl.BlockSpec((hid, hid2), lambda i: (0, 0)),
            pl.BlockSpec((1, hid2), lambda i: (0, 0)),
            pl.BlockSpec((hid2, 1), lambda i: (0, 0)),
            pl.BlockSpec((1, 1), lambda i: (0, 0)),
        ],
        out_specs=[
            pl.BlockSpec((et, hid), lambda i: (i, 0)),
            pl.BlockSpec((1, 1, et), lambda i: (i, 0, 0)),
            pl.BlockSpec((1, 1), lambda i: (0, 0)),
        ],
        out_shape=[
            jax.ShapeDtypeStruct((e, hid), F32),
            jax.ShapeDtypeStruct((e_steps, 1, et), F32),
            jax.ShapeDtypeStruct((1, 1), F32),
        ],
    )(edge_attr, em_w1, em_b1.reshape(1, hid), em_w2, em_b2.reshape(1, hid),
      ea_w1, ea_b1.reshape(1, hid2), ea_w2, ea_b2.reshape(1, 1))

    w_e, s_sum = pl.pallas_call(
        _expsum_body,
        grid=(1,),
        in_specs=[
            pl.BlockSpec((e_steps, et), lambda i: (0, 0)),
            pl.BlockSpec((1, 1), lambda i: (0, 0)),
        ],
        out_specs=[
            pl.BlockSpec((e_steps, et), lambda i: (0, 0)),
            pl.BlockSpec((1, 1), lambda i: (0, 0)),
        ],
        out_shape=[
            jax.ShapeDtypeStruct((e_steps, et), F32),
            jax.ShapeDtypeStruct((1, 1), F32),
        ],
    )(logits.reshape(e_steps, et), lmax)

    src = edge_index[0]
    dst = edge_index[1]
    numer, denom = _run_sc_conv(xl, src, dst, u, v, m16, n, e, hc, heads)
    agg = _run_sc_agg(pe, src, w_e.reshape(e), n, e, hc)

    out = pl.pallas_call(
        functools.partial(_combine_body, n_steps=n_steps),
        grid=(n_steps,),
        in_specs=[
            pl.BlockSpec((_NC, nt, hc), lambda i: (0, i, 0)),
            pl.BlockSpec((_NC, nt, 16), lambda i: (0, i, 0)),
            pl.BlockSpec((_NC, nt, hc), lambda i: (0, i, 0)),
            pl.BlockSpec((nt, hc), lambda i: (i, 0)),
            pl.BlockSpec((nt, 16), lambda i: (i, 0)),
            pl.BlockSpec((nt, 16), lambda i: (i, 0)),
            pl.BlockSpec((1, 4 * heads), lambda i: (0, 0)),
            pl.BlockSpec((1, hc), lambda i: (0, 0)),
            pl.BlockSpec((1, 1), lambda i: (0, 0)),
            pl.BlockSpec((heads, hc), lambda i: (0, 0)),
            pl.BlockSpec((nt, 1), lambda i: (i, 0)),
            pl.BlockSpec((hc, out_d), lambda i: (0, 0)),
            pl.BlockSpec((1, out_d), lambda i: (0, 0)),
        ],
        out_specs=pl.BlockSpec((g, out_d), lambda i: (0, 0)),
        out_shape=jax.ShapeDtypeStruct((g, out_d), F32),
        scratch_shapes=[
            pltpu.VMEM((g, hc), F32),
            pltpu.VMEM((1, g), F32),
        ],
    )(numer, denom, agg, xl, u, v, m16, b_gat.reshape(1, hc), s_sum, rmat,
      batch.reshape(n, 1), fc_w, fc_b.reshape(1, out_d))

    return out.reshape(-1)
